# baseline (speedup 1.0000x reference)
import functools

import jax
import jax.numpy as jnp
from jax.experimental import pallas as pl
from jax.experimental.pallas import tpu as pltpu

_B = 8
_N = 2048
_K = 64
_BN_C = 1.0 / (1.0 + 1e-5) ** 0.5


_M0 = _N // 2
_M1 = _M0 // 4


def _fps_levels(px, py, pz, n, m, outx_ref, outy_ref, outz_ref):
    iota = jax.lax.broadcasted_iota(jnp.int32, (_B, n), 1)
    biota = jax.lax.broadcasted_iota(jnp.int32, (_B, 128), 1)
    zbuf = jnp.zeros((_B, 128), jnp.float32)

    def inner(t, st):
        bufx, bufy, bufz, curx, cury, curz, dists = st
        hit = biota == t
        bufx = jnp.where(hit, curx, bufx)
        bufy = jnp.where(hit, cury, bufy)
        bufz = jnp.where(hit, curz, bufz)
        d = (px - curx) ** 2 + (py - cury) ** 2 + (pz - curz) ** 2
        dn = jnp.minimum(dists, d)
        v = jnp.max(dn, axis=1, keepdims=True)
        idx = jnp.min(jnp.where(dn >= v, iota, n), axis=1, keepdims=True)
        sel = iota == idx
        nx = jnp.sum(jnp.where(sel, px, 0.0), axis=1, keepdims=True)
        ny = jnp.sum(jnp.where(sel, py, 0.0), axis=1, keepdims=True)
        nz = jnp.sum(jnp.where(sel, pz, 0.0), axis=1, keepdims=True)
        return (bufx, bufy, bufz, nx, ny, nz, dn)

    def outer(j, st):
        st = (zbuf, zbuf, zbuf) + st
        bufx, bufy, bufz, curx, cury, curz, dists = jax.lax.fori_loop(
            0, 128, inner, st, unroll=2)
        off = pl.multiple_of(j * 128, 128)
        outx_ref[:, pl.ds(off, 128)] = bufx
        outy_ref[:, pl.ds(off, 128)] = bufy
        outz_ref[:, pl.ds(off, 128)] = bufz
        return (curx, cury, curz, dists)

    d0 = jnp.full((_B, n), jnp.inf, jnp.float32)
    jax.lax.fori_loop(0, m // 128, outer,
                      (px[:, 0:1], py[:, 0:1], pz[:, 0:1], d0))


def _fps_body(px_ref, py_ref, pz_ref,
              cx0_ref, cy0_ref, cz0_ref, cx1_ref, cy1_ref, cz1_ref):
    _fps_levels(px_ref[...], py_ref[...], pz_ref[...], _N, _M0,
                cx0_ref, cy0_ref, cz0_ref)
    _fps_levels(cx0_ref[...], cy0_ref[...], cz0_ref[...], _M0, _M1,
                cx1_ref, cy1_ref, cz1_ref)


def _fps_pallas(pos_b):
    px = pos_b[:, :, 0]
    py = pos_b[:, :, 1]
    pz = pos_b[:, :, 2]
    sds = jax.ShapeDtypeStruct
    outs = pl.pallas_call(
        _fps_body,
        out_shape=(sds((_B, _M0), jnp.float32),) * 3
        + (sds((_B, _M1), jnp.float32),) * 3,
    )(px, py, pz)
    centers0 = jnp.stack(outs[:3], axis=-1)
    centers1 = jnp.stack(outs[3:], axis=-1)
    return centers0, centers1


def _radius_topk_jax(pos_c, centers, r, k):
    m = centers.shape[0]
    idx = jnp.broadcast_to(jnp.arange(k, dtype=jnp.int32)[None, :], (m, k))
    return idx, jnp.ones((m, k), bool)


def _pair_conv_body(tm, k, feat_ref, valid_ref,
                    w1_ref, b1_ref, s1_ref, t1_ref,
                    w2_ref, b2_ref, s2_ref, t2_ref, out_ref):
    x = feat_ref[...]
    z1 = jnp.dot(x, w1_ref[...], preferred_element_type=jnp.float32)
    h1 = jnp.maximum(z1 + b1_ref[...], 0.0) * s1_ref[...] + t1_ref[...]
    z2 = jnp.dot(h1, w2_ref[...], preferred_element_type=jnp.float32)
    h2 = jnp.maximum(z2 + b2_ref[...], 0.0) * s2_ref[...] + t2_ref[...]
    c2 = h2.shape[-1]
    h3 = h2.reshape(tm, k, c2)
    msk = valid_ref[...][:, :, None] > 0
    h3 = jnp.where(msk, h3, -jnp.inf)
    out_ref[...] = jnp.max(h3, axis=1)


def _pair_conv(feat, valid, layers, tm):
    mk, cin = feat.shape
    m = mk // _K
    (w1, b1, s1, t1), (w2, b2, s2, t2) = layers
    c1, c2 = w1.shape[1], w2.shape[1]
    grid = (m // tm,)
    return pl.pallas_call(
        functools.partial(_pair_conv_body, tm, _K),
        grid=grid,
        in_specs=[
            pl.BlockSpec((tm * _K, cin), lambda i: (i, 0)),
            pl.BlockSpec((tm, _K), lambda i: (i, 0)),
            pl.BlockSpec(w1.shape, lambda i: (0, 0)),
            pl.BlockSpec((1, c1), lambda i: (0, 0)),
            pl.BlockSpec((1, c1), lambda i: (0, 0)),
            pl.BlockSpec((1, c1), lambda i: (0, 0)),
            pl.BlockSpec(w2.shape, lambda i: (0, 0)),
            pl.BlockSpec((1, c2), lambda i: (0, 0)),
            pl.BlockSpec((1, c2), lambda i: (0, 0)),
            pl.BlockSpec((1, c2), lambda i: (0, 0)),
        ],
        out_specs=pl.BlockSpec((tm, c2), lambda i: (i, 0)),
        out_shape=jax.ShapeDtypeStruct((m, c2), jnp.float32),
    )(feat, valid, w1, b1, s1, t1, w2, b2, s2, t2)


def _prep_layer(lyr):
    c = lyr['W'].shape[1]
    return (lyr['W'], lyr['b'].reshape(1, c),
            (lyr['gamma'] * _BN_C).reshape(1, c), lyr['beta'].reshape(1, c))


def _global_tail_body(feat_ref, wg_ref, bg_ref, sg_ref, tg_ref,
                      w0_ref, b0_ref, w1_ref, b1_ref, out_ref):
    x = feat_ref[...]
    z = jnp.dot(x, wg_ref[...], preferred_element_type=jnp.float32)
    g = jnp.maximum(z + bg_ref[...], 0.0) * sg_ref[...] + tg_ref[...]
    cg = g.shape[-1]
    g = jnp.max(g.reshape(_B, -1, cg), axis=1)
    g = jnp.maximum(g, 0.0)
    h = jnp.maximum(jnp.dot(g, w0_ref[...], preferred_element_type=jnp.float32)
                    + b0_ref[...], 0.0)
    out_ref[...] = jnp.dot(h, w1_ref[...],
                           preferred_element_type=jnp.float32) + b1_ref[...]


def _global_tail(feat, params):
    wg, bg, sg, tg = _prep_layer(params['mlpg'][0])
    w0 = params['lin0']['W']
    b0 = params['lin0']['b'].reshape(1, -1)
    w1 = params['lin1']['W']
    b1 = params['lin1']['b'].reshape(1, -1)
    nout = w1.shape[1]
    full = lambda a: pl.BlockSpec(a.shape, lambda: (0,) * a.ndim)
    return pl.pallas_call(
        _global_tail_body,
        in_specs=[full(feat), full(wg), full(bg), full(sg), full(tg),
                  full(w0), full(b0), full(w1), full(b1)],
        out_specs=pl.BlockSpec((_B, nout), lambda: (0, 0)),
        out_shape=jax.ShapeDtypeStruct((_B, nout), jnp.float32),
    )(feat, wg, bg, sg, tg, w0, b0, w1, b1)


def kernel(pos, batch, params):
    del batch
    pos_b = pos.reshape(_B, _N, 3)

    m0 = _M0
    centers0, centers1 = _fps_pallas(pos_b)
    idx0, valid0 = jax.vmap(
        lambda pc, c: _radius_topk_jax(pc, c, 0.2, _K))(pos_b, centers0)
    idx0f = idx0.reshape(_B, m0 * _K)
    nbr0 = jnp.stack(
        [jnp.take_along_axis(pos_b[:, :, c], idx0f, axis=1)
         for c in range(3)], axis=-1).reshape(_B, m0, _K, 3)
    rel0 = nbr0 - centers0[:, :, None, :]
    feat0 = rel0.reshape(_B * m0 * _K, 3)
    x1 = _pair_conv(feat0, valid0.reshape(_B * m0, _K).astype(jnp.int32),
                    [_prep_layer(l) for l in params['mlp0']], tm=64)
    x1 = x1.reshape(_B, m0, -1)

    m1 = _M1
    pos1 = centers0
    idx1, valid1 = jax.vmap(
        lambda pc, c: _radius_topk_jax(pc, c, 0.4, _K))(pos1, centers1)
    idx1f = idx1.reshape(_B, m1 * _K)
    nbrp = jnp.stack(
        [jnp.take_along_axis(pos1[:, :, c], idx1f, axis=1)
         for c in range(3)], axis=-1).reshape(_B, m1, _K, 3)
    rel1 = nbrp - centers1[:, :, None, :]
    xg = jnp.take_along_axis(x1[:, :, None, :], idx1[:, :, :, None], axis=1)
    feat1 = jnp.concatenate([xg, rel1], axis=-1).reshape(_B * m1 * _K, -1)
    x2 = _pair_conv(feat1, valid1.reshape(_B * m1, _K).astype(jnp.int32),
                    [_prep_layer(l) for l in params['mlp1']], tm=32)
    x2 = x2.reshape(_B, m1, -1)

    featg = jnp.concatenate([x2, centers1], axis=-1).reshape(_B * m1, -1)
    return _global_tail(featg, params)


# speedup vs baseline: 5.8695x; 5.8695x over previous
import functools

import jax
import jax.numpy as jnp
from jax import lax
from jax.experimental import pallas as pl
from jax.experimental.pallas import tpu as pltpu
from jax.experimental.pallas import tpu_sc as plsc

_B = 8
_N = 2048
_K = 64
_BN_C = 1.0 / (1.0 + 1e-5) ** 0.5


_M0 = _N // 2
_M1 = _M0 // 4


def _fps_levels(px, py, pz, n, m, outx_ref, outy_ref, outz_ref):
    iota = jax.lax.broadcasted_iota(jnp.int32, (_B, n), 1)
    biota = jax.lax.broadcasted_iota(jnp.int32, (_B, 128), 1)
    zbuf = jnp.zeros((_B, 128), jnp.float32)

    def inner(t, st):
        bufx, bufy, bufz, curx, cury, curz, dists = st
        hit = biota == t
        bufx = jnp.where(hit, curx, bufx)
        bufy = jnp.where(hit, cury, bufy)
        bufz = jnp.where(hit, curz, bufz)
        d = (px - curx) ** 2 + (py - cury) ** 2 + (pz - curz) ** 2
        dn = jnp.minimum(dists, d)
        v = jnp.max(dn, axis=1, keepdims=True)
        idx = jnp.min(jnp.where(dn >= v, iota, n), axis=1, keepdims=True)
        sel = iota == idx
        nx = jnp.sum(jnp.where(sel, px, 0.0), axis=1, keepdims=True)
        ny = jnp.sum(jnp.where(sel, py, 0.0), axis=1, keepdims=True)
        nz = jnp.sum(jnp.where(sel, pz, 0.0), axis=1, keepdims=True)
        return (bufx, bufy, bufz, nx, ny, nz, dn)

    def outer(j, st):
        st = (zbuf, zbuf, zbuf) + st
        bufx, bufy, bufz, curx, cury, curz, dists = jax.lax.fori_loop(
            0, 128, inner, st, unroll=2)
        off = pl.multiple_of(j * 128, 128)
        outx_ref[:, pl.ds(off, 128)] = bufx
        outy_ref[:, pl.ds(off, 128)] = bufy
        outz_ref[:, pl.ds(off, 128)] = bufz
        return (curx, cury, curz, dists)

    d0 = jnp.full((_B, n), jnp.inf, jnp.float32)
    jax.lax.fori_loop(0, m // 128, outer,
                      (px[:, 0:1], py[:, 0:1], pz[:, 0:1], d0))


def _fps_body(px_ref, py_ref, pz_ref,
              cx0_ref, cy0_ref, cz0_ref, cx1_ref, cy1_ref, cz1_ref):
    _fps_levels(px_ref[...], py_ref[...], pz_ref[...], _N, _M0,
                cx0_ref, cy0_ref, cz0_ref)
    _fps_levels(cx0_ref[...], cy0_ref[...], cz0_ref[...], _M0, _M1,
                cx1_ref, cy1_ref, cz1_ref)


def _fps_planes(px, py, pz):
    sds = jax.ShapeDtypeStruct
    return pl.pallas_call(
        _fps_body,
        out_shape=(sds((_B, _M0), jnp.float32),) * 3
        + (sds((_B, _M1), jnp.float32),) * 3,
    )(px, py, pz)


def _radius_topk_planes(px, py, pz, cx, cy, cz, r):
    d2 = ((cx[:, :, None] - px[:, None, :]) ** 2
          + (cy[:, :, None] - py[:, None, :]) ** 2
          + (cz[:, :, None] - pz[:, None, :]) ** 2)
    neg = jnp.where(d2 <= r * r, -d2, -jnp.inf)
    vals, idx = jax.lax.top_k(neg, _K)
    return idx.astype(jnp.int32), vals > -jnp.inf


_NC, _NS = 2, 16
_NW = _NC * _NS


def _sc_mesh():
    return plsc.VectorSubcoreMesh(core_axis_name="c", subcore_axis_name="s",
                                  num_cores=_NC, num_subcores=_NS)


def _sc_rel3(px, py, pz, cx, cy, cz, idx_flat, n, m):
    mk = m * _K
    npairs = _B * mk
    P = npairs // _NW
    QC = _NW // _B

    @functools.partial(
        pl.kernel,
        out_type=jax.ShapeDtypeStruct((npairs * 3,), jnp.float32),
        mesh=_sc_mesh(),
        compiler_params=pltpu.CompilerParams(needs_layout_passes=False),
        scratch_types=[pltpu.VMEM((n,), jnp.float32)] * 3
        + [pltpu.VMEM((m,), jnp.float32)] * 3
        + [pltpu.VMEM((P,), jnp.int32), pltpu.VMEM((3 * P,), jnp.float32)],
    )
    def k(px_h, py_h, pz_h, cx_h, cy_h, cz_h, idx_h, out_h,
          tx, ty, tz, tcx, tcy, tcz, ib, ob):
        wid = lax.axis_index("s") * _NC + lax.axis_index("c")
        b = wid // QC
        off = (wid % QC) * P
        pltpu.sync_copy(px_h.at[b], tx)
        pltpu.sync_copy(py_h.at[b], ty)
        pltpu.sync_copy(pz_h.at[b], tz)
        pltpu.sync_copy(cx_h.at[b], tcx)
        pltpu.sync_copy(cy_h.at[b], tcy)
        pltpu.sync_copy(cz_h.at[b], tcz)
        pltpu.sync_copy(idx_h.at[pl.ds(b * mk + off, P)], ib)
        iota = lax.iota(jnp.int32, 16)

        def step(i, carry):
            base = i * 16
            iv = ib[pl.ds(base, 16)]
            ci = lax.shift_right_logical(off + base + iota, 6)
            j3 = (base + iota) * 3
            gx = plsc.load_gather(tx, [iv]) - plsc.load_gather(tcx, [ci])
            plsc.store_scatter(ob, [j3], gx)
            gy = plsc.load_gather(ty, [iv]) - plsc.load_gather(tcy, [ci])
            plsc.store_scatter(ob, [j3 + 1], gy)
            gz = plsc.load_gather(tz, [iv]) - plsc.load_gather(tcz, [ci])
            plsc.store_scatter(ob, [j3 + 2], gz)
            return carry

        lax.fori_loop(0, P // 16, step, 0)
        pltpu.sync_copy(ob, out_h.at[pl.ds((b * mk + off) * 3, 3 * P)])

    return k(px, py, pz, cx, cy, cz, idx_flat).reshape(npairs, 3)


def _sc_rowgather(table, idxg):
    rows_total = idxg.shape[0]
    d = table.shape[1]
    rw = rows_total // _NW
    ch = 512

    @functools.partial(
        pl.kernel,
        out_type=jax.ShapeDtypeStruct((rows_total, d), jnp.float32),
        mesh=_sc_mesh(),
        compiler_params=pltpu.CompilerParams(needs_layout_passes=False),
        scratch_types=[pltpu.VMEM((ch,), jnp.int32),
                       pltpu.VMEM((ch, d), jnp.float32),
                       pltpu.SemaphoreType.DMA],
    )
    def k(tab_h, idx_h, out_h, ibc, rows, sem):
        wid = lax.axis_index("s") * _NC + lax.axis_index("c")
        base = wid * rw

        def step(j, carry):
            r0 = base + j * ch
            pltpu.sync_copy(idx_h.at[pl.ds(r0, ch)], ibc)
            pltpu.async_copy(tab_h.at[ibc], rows, sem).wait()
            pltpu.sync_copy(rows, out_h.at[pl.ds(r0, ch)])
            return carry

        lax.fori_loop(0, rw // ch, step, 0)

    return k(table, idxg)


def _pair_conv_body(tm, k, nf, *refs):
    feats = refs[:nf]
    w1s = refs[nf:2 * nf]
    (valid_ref, b1_ref, s1_ref, t1_ref,
     w2_ref, b2_ref, s2_ref, t2_ref, out_ref) = refs[2 * nf:]
    z1 = b1_ref[...]
    for f, w in zip(feats, w1s):
        z1 = z1 + jnp.dot(f[...], w[...], preferred_element_type=jnp.float32)
    h1 = jnp.maximum(z1, 0.0) * s1_ref[...] + t1_ref[...]
    z2 = jnp.dot(h1, w2_ref[...], preferred_element_type=jnp.float32)
    h2 = jnp.maximum(z2 + b2_ref[...], 0.0) * s2_ref[...] + t2_ref[...]
    c2 = h2.shape[-1]
    h3 = h2.reshape(tm, k, c2)
    msk = valid_ref[...][:, :, None] > 0
    h3 = jnp.where(msk, h3, -jnp.inf)
    out_ref[...] = jnp.max(h3, axis=1)


def _pair_conv(feats_w1, valid, l1, l2, tm):
    mk = feats_w1[0][0].shape[0]
    m = mk // _K
    nf = len(feats_w1)
    b1, s1, t1 = l1
    w2, b2, s2, t2 = l2
    c1, c2 = w2.shape[0], w2.shape[1]
    grid = (m // tm,)
    row_spec = lambda a: pl.BlockSpec((tm * _K, a.shape[1]), lambda i: (i, 0))
    full_spec = lambda a: pl.BlockSpec(a.shape, lambda i: (0, 0))
    in_specs = ([row_spec(f) for f, _ in feats_w1]
                + [full_spec(w) for _, w in feats_w1]
                + [pl.BlockSpec((tm, _K), lambda i: (i, 0)),
                   full_spec(b1), full_spec(s1), full_spec(t1),
                   full_spec(w2), full_spec(b2), full_spec(s2),
                   full_spec(t2)])
    args = ([f for f, _ in feats_w1] + [w for _, w in feats_w1]
            + [valid, b1, s1, t1, w2, b2, s2, t2])
    return pl.pallas_call(
        functools.partial(_pair_conv_body, tm, _K, nf),
        grid=grid,
        in_specs=in_specs,
        out_specs=pl.BlockSpec((tm, c2), lambda i: (i, 0)),
        out_shape=jax.ShapeDtypeStruct((m, c2), jnp.float32),
    )(*args)


def _prep_layer(lyr):
    c = lyr['W'].shape[1]
    return (lyr['W'], lyr['b'].reshape(1, c),
            (lyr['gamma'] * _BN_C).reshape(1, c), lyr['beta'].reshape(1, c))


def _global_tail_body(feat_ref, wg_ref, bg_ref, sg_ref, tg_ref,
                      w0_ref, b0_ref, w1_ref, b1_ref, out_ref):
    x = feat_ref[...]
    z = jnp.dot(x, wg_ref[...], preferred_element_type=jnp.float32)
    g = jnp.maximum(z + bg_ref[...], 0.0) * sg_ref[...] + tg_ref[...]
    cg = g.shape[-1]
    g = jnp.max(g.reshape(_B, -1, cg), axis=1)
    g = jnp.maximum(g, 0.0)
    h = jnp.maximum(jnp.dot(g, w0_ref[...], preferred_element_type=jnp.float32)
                    + b0_ref[...], 0.0)
    out_ref[...] = jnp.dot(h, w1_ref[...],
                           preferred_element_type=jnp.float32) + b1_ref[...]


def _global_tail(feat, params):
    wg, bg, sg, tg = _prep_layer(params['mlpg'][0])
    w0 = params['lin0']['W']
    b0 = params['lin0']['b'].reshape(1, -1)
    w1 = params['lin1']['W']
    b1 = params['lin1']['b'].reshape(1, -1)
    nout = w1.shape[1]
    full = lambda a: pl.BlockSpec(a.shape, lambda: (0,) * a.ndim)
    return pl.pallas_call(
        _global_tail_body,
        in_specs=[full(feat), full(wg), full(bg), full(sg), full(tg),
                  full(w0), full(b0), full(w1), full(b1)],
        out_specs=pl.BlockSpec((_B, nout), lambda: (0, 0)),
        out_shape=jax.ShapeDtypeStruct((_B, nout), jnp.float32),
    )(feat, wg, bg, sg, tg, w0, b0, w1, b1)


def kernel(pos, batch, params):
    del batch
    pos_b = pos.reshape(_B, _N, 3)
    px, py, pz = pos_b[:, :, 0], pos_b[:, :, 1], pos_b[:, :, 2]
    cx0, cy0, cz0, cx1, cy1, cz1 = _fps_planes(px, py, pz)

    idx0, valid0 = _radius_topk_planes(px, py, pz, cx0, cy0, cz0, 0.2)
    feat0 = _sc_rel3(px, py, pz, cx0, cy0, cz0, idx0.reshape(-1), _N, _M0)
    (w1, b1, s1, t1), (w2, b2, s2, t2) = [
        _prep_layer(l) for l in params['mlp0']]
    x1 = _pair_conv([(feat0, w1)],
                    valid0.reshape(_B * _M0, _K).astype(jnp.int32),
                    (b1, s1, t1), (w2, b2, s2, t2), tm=64)

    idx1, valid1 = _radius_topk_planes(cx0, cy0, cz0, cx1, cy1, cz1, 0.4)
    rel1 = _sc_rel3(cx0, cy0, cz0, cx1, cy1, cz1, idx1.reshape(-1),
                    _M0, _M1)
    idxg = (idx1.reshape(_B, -1)
            + (jnp.arange(_B, dtype=jnp.int32) * _M0)[:, None]).reshape(-1)
    xg = _sc_rowgather(x1, idxg)
    (w1b, b1b, s1b, t1b), (w2b, b2b, s2b, t2b) = [
        _prep_layer(l) for l in params['mlp1']]
    x2 = _pair_conv([(xg, w1b[:-3]), (rel1, w1b[-3:])],
                    valid1.reshape(_B * _M1, _K).astype(jnp.int32),
                    (b1b, s1b, t1b), (w2b, b2b, s2b, t2b), tm=32)
    x2 = x2.reshape(_B, _M1, -1)

    centers1 = jnp.stack([cx1, cy1, cz1], axis=-1)
    featg = jnp.concatenate([x2, centers1], axis=-1).reshape(_B * _M1, -1)
    return _global_tail(featg, params)


# speedup vs baseline: 5.8857x; 1.0028x over previous
import functools

import jax
import jax.numpy as jnp
from jax import lax
from jax.experimental import pallas as pl
from jax.experimental.pallas import tpu as pltpu
from jax.experimental.pallas import tpu_sc as plsc

_B = 8
_N = 2048
_K = 64
_BN_C = 1.0 / (1.0 + 1e-5) ** 0.5


_M0 = _N // 2
_M1 = _M0 // 4


def _fps_levels(px, py, pz, n, m, outx_ref, outy_ref, outz_ref, d_ref):
    iota = jax.lax.broadcasted_iota(jnp.int32, (_B, n), 1)
    biota = jax.lax.broadcasted_iota(jnp.int32, (_B, 128), 1)
    zbuf = jnp.zeros((_B, 128), jnp.float32)

    def inner(t, st):
        bufx, bufy, bufz, curx, cury, curz = st
        hit = biota == t
        bufx = jnp.where(hit, curx, bufx)
        bufy = jnp.where(hit, cury, bufy)
        bufz = jnp.where(hit, curz, bufz)
        d = (px - curx) ** 2 + (py - cury) ** 2 + (pz - curz) ** 2
        dn = jnp.minimum(d_ref[:, :n], d)
        d_ref[:, :n] = dn
        v = jnp.max(dn, axis=1, keepdims=True)
        idx = jnp.min(jnp.where(dn >= v, iota, n), axis=1, keepdims=True)
        sel = iota == idx
        nx = jnp.sum(jnp.where(sel, px, 0.0), axis=1, keepdims=True)
        ny = jnp.sum(jnp.where(sel, py, 0.0), axis=1, keepdims=True)
        nz = jnp.sum(jnp.where(sel, pz, 0.0), axis=1, keepdims=True)
        return (bufx, bufy, bufz, nx, ny, nz)

    def outer(j, st):
        st = (zbuf, zbuf, zbuf) + st
        bufx, bufy, bufz, curx, cury, curz = jax.lax.fori_loop(
            0, 128, inner, st, unroll=2)
        off = pl.multiple_of(j * 128, 128)
        outx_ref[:, pl.ds(off, 128)] = bufx
        outy_ref[:, pl.ds(off, 128)] = bufy
        outz_ref[:, pl.ds(off, 128)] = bufz
        return (curx, cury, curz)

    d_ref[:, :n] = jnp.full((_B, n), jnp.inf, jnp.float32)
    jax.lax.fori_loop(0, m // 128, outer,
                      (px[:, 0:1], py[:, 0:1], pz[:, 0:1]))


def _fps_body(px_ref, py_ref, pz_ref,
              cx0_ref, cy0_ref, cz0_ref, cx1_ref, cy1_ref, cz1_ref,
              d_ref):
    _fps_levels(px_ref[...], py_ref[...], pz_ref[...], _N, _M0,
                cx0_ref, cy0_ref, cz0_ref, d_ref)
    _fps_levels(cx0_ref[...], cy0_ref[...], cz0_ref[...], _M0, _M1,
                cx1_ref, cy1_ref, cz1_ref, d_ref)


def _fps_planes(px, py, pz):
    sds = jax.ShapeDtypeStruct
    return pl.pallas_call(
        _fps_body,
        out_shape=(sds((_B, _M0), jnp.float32),) * 3
        + (sds((_B, _M1), jnp.float32),) * 3,
        scratch_shapes=[pltpu.VMEM((_B, _N), jnp.float32)],
    )(px, py, pz)


def _radius_topk_planes(px, py, pz, cx, cy, cz, r):
    d2 = ((cx[:, :, None] - px[:, None, :]) ** 2
          + (cy[:, :, None] - py[:, None, :]) ** 2
          + (cz[:, :, None] - pz[:, None, :]) ** 2)
    neg = jnp.where(d2 <= r * r, -d2, -jnp.inf)
    vals, idx = jax.lax.top_k(neg, _K)
    return idx.astype(jnp.int32), vals > -jnp.inf


_NC, _NS = 2, 16
_NW = _NC * _NS


def _sc_mesh():
    return plsc.VectorSubcoreMesh(core_axis_name="c", subcore_axis_name="s",
                                  num_cores=_NC, num_subcores=_NS)


def _sc_rel3(px, py, pz, cx, cy, cz, idx_flat, n, m):
    mk = m * _K
    npairs = _B * mk
    P = npairs // _NW
    QC = _NW // _B

    @functools.partial(
        pl.kernel,
        out_type=jax.ShapeDtypeStruct((npairs * 3,), jnp.float32),
        mesh=_sc_mesh(),
        compiler_params=pltpu.CompilerParams(needs_layout_passes=False),
        scratch_types=[pltpu.VMEM((n,), jnp.float32)] * 3
        + [pltpu.VMEM((m,), jnp.float32)] * 3
        + [pltpu.VMEM((P,), jnp.int32), pltpu.VMEM((3 * P,), jnp.float32)],
    )
    def k(px_h, py_h, pz_h, cx_h, cy_h, cz_h, idx_h, out_h,
          tx, ty, tz, tcx, tcy, tcz, ib, ob):
        wid = lax.axis_index("s") * _NC + lax.axis_index("c")
        b = wid // QC
        off = (wid % QC) * P
        pltpu.sync_copy(px_h.at[b], tx)
        pltpu.sync_copy(py_h.at[b], ty)
        pltpu.sync_copy(pz_h.at[b], tz)
        pltpu.sync_copy(cx_h.at[b], tcx)
        pltpu.sync_copy(cy_h.at[b], tcy)
        pltpu.sync_copy(cz_h.at[b], tcz)
        pltpu.sync_copy(idx_h.at[pl.ds(b * mk + off, P)], ib)
        iota = lax.iota(jnp.int32, 16)

        def step(i, carry):
            base = i * 16
            iv = ib[pl.ds(base, 16)]
            ci = lax.shift_right_logical(off + base + iota, 6)
            j3 = (base + iota) * 3
            gx = plsc.load_gather(tx, [iv]) - plsc.load_gather(tcx, [ci])
            plsc.store_scatter(ob, [j3], gx)
            gy = plsc.load_gather(ty, [iv]) - plsc.load_gather(tcy, [ci])
            plsc.store_scatter(ob, [j3 + 1], gy)
            gz = plsc.load_gather(tz, [iv]) - plsc.load_gather(tcz, [ci])
            plsc.store_scatter(ob, [j3 + 2], gz)
            return carry

        lax.fori_loop(0, P // 16, step, 0)
        pltpu.sync_copy(ob, out_h.at[pl.ds((b * mk + off) * 3, 3 * P)])

    return k(px, py, pz, cx, cy, cz, idx_flat).reshape(npairs, 3)


def _sc_rowgather(table, idxg):
    rows_total = idxg.shape[0]
    d = table.shape[1]
    rw = rows_total // _NW
    ch = 512

    @functools.partial(
        pl.kernel,
        out_type=jax.ShapeDtypeStruct((rows_total, d), jnp.float32),
        mesh=_sc_mesh(),
        compiler_params=pltpu.CompilerParams(needs_layout_passes=False),
        scratch_types=[pltpu.VMEM((ch,), jnp.int32),
                       pltpu.VMEM((ch, d), jnp.float32),
                       pltpu.SemaphoreType.DMA],
    )
    def k(tab_h, idx_h, out_h, ibc, rows, sem):
        wid = lax.axis_index("s") * _NC + lax.axis_index("c")
        base = wid * rw

        def step(j, carry):
            r0 = base + j * ch
            pltpu.sync_copy(idx_h.at[pl.ds(r0, ch)], ibc)
            pltpu.async_copy(tab_h.at[ibc], rows, sem).wait()
            pltpu.sync_copy(rows, out_h.at[pl.ds(r0, ch)])
            return carry

        lax.fori_loop(0, rw // ch, step, 0)

    return k(table, idxg)


def _pair_conv_body(tm, k, nf, *refs):
    feats = refs[:nf]
    w1s = refs[nf:2 * nf]
    (valid_ref, b1_ref, s1_ref, t1_ref,
     w2_ref, b2_ref, s2_ref, t2_ref, out_ref) = refs[2 * nf:]
    z1 = b1_ref[...]
    for f, w in zip(feats, w1s):
        z1 = z1 + jnp.dot(f[...], w[...], preferred_element_type=jnp.float32)
    h1 = jnp.maximum(z1, 0.0) * s1_ref[...] + t1_ref[...]
    z2 = jnp.dot(h1, w2_ref[...], preferred_element_type=jnp.float32)
    h2 = jnp.maximum(z2 + b2_ref[...], 0.0) * s2_ref[...] + t2_ref[...]
    c2 = h2.shape[-1]
    h3 = h2.reshape(tm, k, c2)
    msk = valid_ref[...][:, :, None] > 0
    h3 = jnp.where(msk, h3, -jnp.inf)
    out_ref[...] = jnp.max(h3, axis=1)


def _pair_conv(feats_w1, valid, l1, l2, tm):
    mk = feats_w1[0][0].shape[0]
    m = mk // _K
    nf = len(feats_w1)
    b1, s1, t1 = l1
    w2, b2, s2, t2 = l2
    c1, c2 = w2.shape[0], w2.shape[1]
    grid = (m // tm,)
    row_spec = lambda a: pl.BlockSpec((tm * _K, a.shape[1]), lambda i: (i, 0))
    full_spec = lambda a: pl.BlockSpec(a.shape, lambda i: (0, 0))
    in_specs = ([row_spec(f) for f, _ in feats_w1]
                + [full_spec(w) for _, w in feats_w1]
                + [pl.BlockSpec((tm, _K), lambda i: (i, 0)),
                   full_spec(b1), full_spec(s1), full_spec(t1),
                   full_spec(w2), full_spec(b2), full_spec(s2),
                   full_spec(t2)])
    args = ([f for f, _ in feats_w1] + [w for _, w in feats_w1]
            + [valid, b1, s1, t1, w2, b2, s2, t2])
    return pl.pallas_call(
        functools.partial(_pair_conv_body, tm, _K, nf),
        grid=grid,
        in_specs=in_specs,
        out_specs=pl.BlockSpec((tm, c2), lambda i: (i, 0)),
        out_shape=jax.ShapeDtypeStruct((m, c2), jnp.float32),
    )(*args)


def _prep_layer(lyr):
    c = lyr['W'].shape[1]
    return (lyr['W'], lyr['b'].reshape(1, c),
            (lyr['gamma'] * _BN_C).reshape(1, c), lyr['beta'].reshape(1, c))


def _global_tail_body(feat_ref, wg_ref, bg_ref, sg_ref, tg_ref,
                      w0_ref, b0_ref, w1_ref, b1_ref, out_ref):
    x = feat_ref[...]
    z = jnp.dot(x, wg_ref[...], preferred_element_type=jnp.float32)
    g = jnp.maximum(z + bg_ref[...], 0.0) * sg_ref[...] + tg_ref[...]
    cg = g.shape[-1]
    g = jnp.max(g.reshape(_B, -1, cg), axis=1)
    g = jnp.maximum(g, 0.0)
    h = jnp.maximum(jnp.dot(g, w0_ref[...], preferred_element_type=jnp.float32)
                    + b0_ref[...], 0.0)
    out_ref[...] = jnp.dot(h, w1_ref[...],
                           preferred_element_type=jnp.float32) + b1_ref[...]


def _global_tail(feat, params):
    wg, bg, sg, tg = _prep_layer(params['mlpg'][0])
    w0 = params['lin0']['W']
    b0 = params['lin0']['b'].reshape(1, -1)
    w1 = params['lin1']['W']
    b1 = params['lin1']['b'].reshape(1, -1)
    nout = w1.shape[1]
    full = lambda a: pl.BlockSpec(a.shape, lambda: (0,) * a.ndim)
    return pl.pallas_call(
        _global_tail_body,
        in_specs=[full(feat), full(wg), full(bg), full(sg), full(tg),
                  full(w0), full(b0), full(w1), full(b1)],
        out_specs=pl.BlockSpec((_B, nout), lambda: (0, 0)),
        out_shape=jax.ShapeDtypeStruct((_B, nout), jnp.float32),
    )(feat, wg, bg, sg, tg, w0, b0, w1, b1)


def kernel(pos, batch, params):
    del batch
    pos_b = pos.reshape(_B, _N, 3)
    px, py, pz = pos_b[:, :, 0], pos_b[:, :, 1], pos_b[:, :, 2]
    cx0, cy0, cz0, cx1, cy1, cz1 = _fps_planes(px, py, pz)

    idx0, valid0 = _radius_topk_planes(px, py, pz, cx0, cy0, cz0, 0.2)
    feat0 = _sc_rel3(px, py, pz, cx0, cy0, cz0, idx0.reshape(-1), _N, _M0)
    (w1, b1, s1, t1), (w2, b2, s2, t2) = [
        _prep_layer(l) for l in params['mlp0']]
    x1 = _pair_conv([(feat0, w1)],
                    valid0.reshape(_B * _M0, _K).astype(jnp.int32),
                    (b1, s1, t1), (w2, b2, s2, t2), tm=64)

    idx1, valid1 = _radius_topk_planes(cx0, cy0, cz0, cx1, cy1, cz1, 0.4)
    rel1 = _sc_rel3(cx0, cy0, cz0, cx1, cy1, cz1, idx1.reshape(-1),
                    _M0, _M1)
    idxg = (idx1.reshape(_B, -1)
            + (jnp.arange(_B, dtype=jnp.int32) * _M0)[:, None]).reshape(-1)
    xg = _sc_rowgather(x1, idxg)
    (w1b, b1b, s1b, t1b), (w2b, b2b, s2b, t2b) = [
        _prep_layer(l) for l in params['mlp1']]
    x2 = _pair_conv([(xg, w1b[:-3]), (rel1, w1b[-3:])],
                    valid1.reshape(_B * _M1, _K).astype(jnp.int32),
                    (b1b, s1b, t1b), (w2b, b2b, s2b, t2b), tm=32)
    x2 = x2.reshape(_B, _M1, -1)

    centers1 = jnp.stack([cx1, cy1, cz1], axis=-1)
    featg = jnp.concatenate([x2, centers1], axis=-1).reshape(_B * _M1, -1)
    return _global_tail(featg, params)


# speedup vs baseline: 5.9124x; 1.0045x over previous
import functools

import jax
import jax.numpy as jnp
from jax import lax
from jax.experimental import pallas as pl
from jax.experimental.pallas import tpu as pltpu
from jax.experimental.pallas import tpu_sc as plsc

_B = 8
_N = 2048
_K = 64
_BN_C = 1.0 / (1.0 + 1e-5) ** 0.5


_M0 = _N // 2
_M1 = _M0 // 4


def _fps_levels(px, py, pz, n, m, outx_ref, outy_ref, outz_ref, d_ref):
    iota = jax.lax.broadcasted_iota(jnp.int32, (_B, n), 1)
    biota = jax.lax.broadcasted_iota(jnp.int32, (_B, 128), 1)
    zbuf = jnp.zeros((_B, 128), jnp.float32)

    def inner(t, st):
        bufx, bufy, bufz, curx, cury, curz = st
        hit = biota == t
        bufx = jnp.where(hit, curx, bufx)
        bufy = jnp.where(hit, cury, bufy)
        bufz = jnp.where(hit, curz, bufz)
        d = (px - curx) ** 2 + (py - cury) ** 2 + (pz - curz) ** 2
        dn = jnp.minimum(d_ref[:, :n], d)
        d_ref[:, :n] = dn
        v = jnp.max(dn, axis=1, keepdims=True)
        idx = jnp.min(jnp.where(dn >= v, iota, n), axis=1, keepdims=True)
        sel = iota == idx
        nx = jnp.sum(jnp.where(sel, px, 0.0), axis=1, keepdims=True)
        ny = jnp.sum(jnp.where(sel, py, 0.0), axis=1, keepdims=True)
        nz = jnp.sum(jnp.where(sel, pz, 0.0), axis=1, keepdims=True)
        return (bufx, bufy, bufz, nx, ny, nz)

    def outer(j, st):
        st = (zbuf, zbuf, zbuf) + st
        bufx, bufy, bufz, curx, cury, curz = jax.lax.fori_loop(
            0, 128, inner, st, unroll=4)
        off = pl.multiple_of(j * 128, 128)
        outx_ref[:, pl.ds(off, 128)] = bufx
        outy_ref[:, pl.ds(off, 128)] = bufy
        outz_ref[:, pl.ds(off, 128)] = bufz
        return (curx, cury, curz)

    d_ref[:, :n] = jnp.full((_B, n), jnp.inf, jnp.float32)
    jax.lax.fori_loop(0, m // 128, outer,
                      (px[:, 0:1], py[:, 0:1], pz[:, 0:1]))


def _fps_body(px_ref, py_ref, pz_ref,
              cx0_ref, cy0_ref, cz0_ref, cx1_ref, cy1_ref, cz1_ref,
              d_ref):
    _fps_levels(px_ref[...], py_ref[...], pz_ref[...], _N, _M0,
                cx0_ref, cy0_ref, cz0_ref, d_ref)
    _fps_levels(cx0_ref[...], cy0_ref[...], cz0_ref[...], _M0, _M1,
                cx1_ref, cy1_ref, cz1_ref, d_ref)


def _fps_planes(px, py, pz):
    sds = jax.ShapeDtypeStruct
    return pl.pallas_call(
        _fps_body,
        out_shape=(sds((_B, _M0), jnp.float32),) * 3
        + (sds((_B, _M1), jnp.float32),) * 3,
        scratch_shapes=[pltpu.VMEM((_B, _N), jnp.float32)],
    )(px, py, pz)


def _radius_topk_planes(px, py, pz, cx, cy, cz, r):
    d2 = ((cx[:, :, None] - px[:, None, :]) ** 2
          + (cy[:, :, None] - py[:, None, :]) ** 2
          + (cz[:, :, None] - pz[:, None, :]) ** 2)
    neg = jnp.where(d2 <= r * r, -d2, -jnp.inf)
    vals, idx = jax.lax.top_k(neg, _K)
    return idx.astype(jnp.int32), vals > -jnp.inf


_NC, _NS = 2, 16
_NW = _NC * _NS


def _sc_mesh():
    return plsc.VectorSubcoreMesh(core_axis_name="c", subcore_axis_name="s",
                                  num_cores=_NC, num_subcores=_NS)


def _sc_rel3(px, py, pz, cx, cy, cz, idx_flat, n, m):
    mk = m * _K
    npairs = _B * mk
    P = npairs // _NW
    QC = _NW // _B

    @functools.partial(
        pl.kernel,
        out_type=jax.ShapeDtypeStruct((npairs * 3,), jnp.float32),
        mesh=_sc_mesh(),
        compiler_params=pltpu.CompilerParams(needs_layout_passes=False),
        scratch_types=[pltpu.VMEM((n,), jnp.float32)] * 3
        + [pltpu.VMEM((m,), jnp.float32)] * 3
        + [pltpu.VMEM((P,), jnp.int32), pltpu.VMEM((3 * P,), jnp.float32)],
    )
    def k(px_h, py_h, pz_h, cx_h, cy_h, cz_h, idx_h, out_h,
          tx, ty, tz, tcx, tcy, tcz, ib, ob):
        wid = lax.axis_index("s") * _NC + lax.axis_index("c")
        b = wid // QC
        off = (wid % QC) * P
        pltpu.sync_copy(px_h.at[b], tx)
        pltpu.sync_copy(py_h.at[b], ty)
        pltpu.sync_copy(pz_h.at[b], tz)
        pltpu.sync_copy(cx_h.at[b], tcx)
        pltpu.sync_copy(cy_h.at[b], tcy)
        pltpu.sync_copy(cz_h.at[b], tcz)
        pltpu.sync_copy(idx_h.at[pl.ds(b * mk + off, P)], ib)
        iota = lax.iota(jnp.int32, 16)

        def step(i, carry):
            base = i * 16
            iv = ib[pl.ds(base, 16)]
            ci = lax.shift_right_logical(off + base + iota, 6)
            j3 = (base + iota) * 3
            gx = plsc.load_gather(tx, [iv]) - plsc.load_gather(tcx, [ci])
            plsc.store_scatter(ob, [j3], gx)
            gy = plsc.load_gather(ty, [iv]) - plsc.load_gather(tcy, [ci])
            plsc.store_scatter(ob, [j3 + 1], gy)
            gz = plsc.load_gather(tz, [iv]) - plsc.load_gather(tcz, [ci])
            plsc.store_scatter(ob, [j3 + 2], gz)
            return carry

        lax.fori_loop(0, P // 16, step, 0)
        pltpu.sync_copy(ob, out_h.at[pl.ds((b * mk + off) * 3, 3 * P)])

    return k(px, py, pz, cx, cy, cz, idx_flat).reshape(npairs, 3)


def _sc_rowgather(table, idxg):
    rows_total = idxg.shape[0]
    d = table.shape[1]
    rw = rows_total // _NW
    ch = 512

    @functools.partial(
        pl.kernel,
        out_type=jax.ShapeDtypeStruct((rows_total, d), jnp.float32),
        mesh=_sc_mesh(),
        compiler_params=pltpu.CompilerParams(needs_layout_passes=False),
        scratch_types=[pltpu.VMEM((ch,), jnp.int32),
                       pltpu.VMEM((ch, d), jnp.float32),
                       pltpu.SemaphoreType.DMA],
    )
    def k(tab_h, idx_h, out_h, ibc, rows, sem):
        wid = lax.axis_index("s") * _NC + lax.axis_index("c")
        base = wid * rw

        def step(j, carry):
            r0 = base + j * ch
            pltpu.sync_copy(idx_h.at[pl.ds(r0, ch)], ibc)
            pltpu.async_copy(tab_h.at[ibc], rows, sem).wait()
            pltpu.sync_copy(rows, out_h.at[pl.ds(r0, ch)])
            return carry

        lax.fori_loop(0, rw // ch, step, 0)

    return k(table, idxg)


def _pair_conv_body(tm, k, nf, *refs):
    feats = refs[:nf]
    w1s = refs[nf:2 * nf]
    (valid_ref, b1_ref, s1_ref, t1_ref,
     w2_ref, b2_ref, s2_ref, t2_ref, out_ref) = refs[2 * nf:]
    z1 = b1_ref[...]
    for f, w in zip(feats, w1s):
        z1 = z1 + jnp.dot(f[...], w[...], preferred_element_type=jnp.float32)
    h1 = jnp.maximum(z1, 0.0) * s1_ref[...] + t1_ref[...]
    z2 = jnp.dot(h1, w2_ref[...], preferred_element_type=jnp.float32)
    h2 = jnp.maximum(z2 + b2_ref[...], 0.0) * s2_ref[...] + t2_ref[...]
    c2 = h2.shape[-1]
    h3 = h2.reshape(tm, k, c2)
    msk = valid_ref[...][:, :, None] > 0
    h3 = jnp.where(msk, h3, -jnp.inf)
    out_ref[...] = jnp.max(h3, axis=1)


def _pair_conv(feats_w1, valid, l1, l2, tm):
    mk = feats_w1[0][0].shape[0]
    m = mk // _K
    nf = len(feats_w1)
    b1, s1, t1 = l1
    w2, b2, s2, t2 = l2
    c1, c2 = w2.shape[0], w2.shape[1]
    grid = (m // tm,)
    row_spec = lambda a: pl.BlockSpec((tm * _K, a.shape[1]), lambda i: (i, 0))
    full_spec = lambda a: pl.BlockSpec(a.shape, lambda i: (0, 0))
    in_specs = ([row_spec(f) for f, _ in feats_w1]
                + [full_spec(w) for _, w in feats_w1]
                + [pl.BlockSpec((tm, _K), lambda i: (i, 0)),
                   full_spec(b1), full_spec(s1), full_spec(t1),
                   full_spec(w2), full_spec(b2), full_spec(s2),
                   full_spec(t2)])
    args = ([f for f, _ in feats_w1] + [w for _, w in feats_w1]
            + [valid, b1, s1, t1, w2, b2, s2, t2])
    return pl.pallas_call(
        functools.partial(_pair_conv_body, tm, _K, nf),
        grid=grid,
        in_specs=in_specs,
        out_specs=pl.BlockSpec((tm, c2), lambda i: (i, 0)),
        out_shape=jax.ShapeDtypeStruct((m, c2), jnp.float32),
    )(*args)


def _prep_layer(lyr):
    c = lyr['W'].shape[1]
    return (lyr['W'], lyr['b'].reshape(1, c),
            (lyr['gamma'] * _BN_C).reshape(1, c), lyr['beta'].reshape(1, c))


def _global_tail_body(feat_ref, wg_ref, bg_ref, sg_ref, tg_ref,
                      w0_ref, b0_ref, w1_ref, b1_ref, out_ref):
    x = feat_ref[...]
    z = jnp.dot(x, wg_ref[...], preferred_element_type=jnp.float32)
    g = jnp.maximum(z + bg_ref[...], 0.0) * sg_ref[...] + tg_ref[...]
    cg = g.shape[-1]
    g = jnp.max(g.reshape(_B, -1, cg), axis=1)
    g = jnp.maximum(g, 0.0)
    h = jnp.maximum(jnp.dot(g, w0_ref[...], preferred_element_type=jnp.float32)
                    + b0_ref[...], 0.0)
    out_ref[...] = jnp.dot(h, w1_ref[...],
                           preferred_element_type=jnp.float32) + b1_ref[...]


def _global_tail(feat, params):
    wg, bg, sg, tg = _prep_layer(params['mlpg'][0])
    w0 = params['lin0']['W']
    b0 = params['lin0']['b'].reshape(1, -1)
    w1 = params['lin1']['W']
    b1 = params['lin1']['b'].reshape(1, -1)
    nout = w1.shape[1]
    full = lambda a: pl.BlockSpec(a.shape, lambda: (0,) * a.ndim)
    return pl.pallas_call(
        _global_tail_body,
        in_specs=[full(feat), full(wg), full(bg), full(sg), full(tg),
                  full(w0), full(b0), full(w1), full(b1)],
        out_specs=pl.BlockSpec((_B, nout), lambda: (0, 0)),
        out_shape=jax.ShapeDtypeStruct((_B, nout), jnp.float32),
    )(feat, wg, bg, sg, tg, w0, b0, w1, b1)


def kernel(pos, batch, params):
    del batch
    pos_b = pos.reshape(_B, _N, 3)
    px, py, pz = pos_b[:, :, 0], pos_b[:, :, 1], pos_b[:, :, 2]
    cx0, cy0, cz0, cx1, cy1, cz1 = _fps_planes(px, py, pz)

    idx0, valid0 = _radius_topk_planes(px, py, pz, cx0, cy0, cz0, 0.2)
    feat0 = _sc_rel3(px, py, pz, cx0, cy0, cz0, idx0.reshape(-1), _N, _M0)
    (w1, b1, s1, t1), (w2, b2, s2, t2) = [
        _prep_layer(l) for l in params['mlp0']]
    x1 = _pair_conv([(feat0, w1)],
                    valid0.reshape(_B * _M0, _K).astype(jnp.int32),
                    (b1, s1, t1), (w2, b2, s2, t2), tm=64)

    idx1, valid1 = _radius_topk_planes(cx0, cy0, cz0, cx1, cy1, cz1, 0.4)
    rel1 = _sc_rel3(cx0, cy0, cz0, cx1, cy1, cz1, idx1.reshape(-1),
                    _M0, _M1)
    idxg = (idx1.reshape(_B, -1)
            + (jnp.arange(_B, dtype=jnp.int32) * _M0)[:, None]).reshape(-1)
    xg = _sc_rowgather(x1, idxg)
    (w1b, b1b, s1b, t1b), (w2b, b2b, s2b, t2b) = [
        _prep_layer(l) for l in params['mlp1']]
    x2 = _pair_conv([(xg, w1b[:-3]), (rel1, w1b[-3:])],
                    valid1.reshape(_B * _M1, _K).astype(jnp.int32),
                    (b1b, s1b, t1b), (w2b, b2b, s2b, t2b), tm=32)
    x2 = x2.reshape(_B, _M1, -1)

    centers1 = jnp.stack([cx1, cy1, cz1], axis=-1)
    featg = jnp.concatenate([x2, centers1], axis=-1).reshape(_B * _M1, -1)
    return _global_tail(featg, params)
